# Initial kernel scaffold; baseline (speedup 1.0000x reference)
import jax
import jax.numpy as jnp
from jax.experimental import pallas as pl


def kernel(x, table):
    raise NotImplementedError("write your pallas kernel here")


# speedup vs baseline: 2.7596x; 2.7596x over previous
import functools

import jax
import jax.numpy as jnp
from jax import lax
from jax.experimental import pallas as pl
from jax.experimental.pallas import tpu as pltpu
from jax.experimental.pallas import tpu_sc as plsc

_BATCH = 4096
_SEQ = 50
_VOCAB = 100000
_D = 128
_B = _BATCH * _SEQ
_NW = 32
_B_PER_W = _B // _NW
_CHUNK = 128
_NCHUNK = _B_PER_W // _CHUNK

_mesh = plsc.VectorSubcoreMesh(core_axis_name="c", subcore_axis_name="s")


@functools.partial(
    pl.kernel,
    mesh=_mesh,
    out_type=jax.ShapeDtypeStruct((_B, _D), jnp.float32),
    scratch_types=[
        pltpu.VMEM((_CHUNK,), jnp.int32),
        pltpu.VMEM((_CHUNK, _D), jnp.float32),
        pltpu.SemaphoreType.DMA,
    ],
)
def _emb_lookup(x_hbm, table_hbm, out_hbm, idx_v, rows_v, sem):
    wid = lax.axis_index("s") * 2 + lax.axis_index("c")
    base = wid * _B_PER_W

    def body(i, carry):
        off = base + i * _CHUNK
        pltpu.sync_copy(x_hbm.at[pl.ds(off, _CHUNK)], idx_v)
        pltpu.async_copy(table_hbm.at[idx_v], rows_v, sem).wait()
        pltpu.sync_copy(rows_v, out_hbm.at[pl.ds(off, _CHUNK)])
        return carry

    lax.fori_loop(0, _NCHUNK, body, 0)


def kernel(x, table):
    out = _emb_lookup(x.reshape(_B), table)
    return out.reshape(_BATCH, _SEQ, _D)


# speedup vs baseline: 3.3073x; 1.1984x over previous
import functools

import jax
import jax.numpy as jnp
from jax import lax
from jax.experimental import pallas as pl
from jax.experimental.pallas import tpu as pltpu
from jax.experimental.pallas import tpu_sc as plsc

_BATCH = 4096
_SEQ = 50
_VOCAB = 100000
_D = 128
_B = _BATCH * _SEQ
_NW = 32
_B_PER_W = _B // _NW
_CHUNK = 128
_NCHUNK = _B_PER_W // _CHUNK
_NBUF = 5

_mesh = plsc.VectorSubcoreMesh(core_axis_name="c", subcore_axis_name="s")


@functools.partial(
    pl.kernel,
    mesh=_mesh,
    out_type=jax.ShapeDtypeStruct((_B, _D), jnp.float32),
    scratch_types=[
        pltpu.VMEM((_NCHUNK, _CHUNK), jnp.int32),
        pltpu.VMEM((_NBUF, _CHUNK, _D), jnp.float32),
        pltpu.SemaphoreType.DMA((_NBUF,)),
    ],
)
def _emb_lookup(x_hbm, table_hbm, out_hbm, idx_v, rows_v, sems):
    wid = lax.axis_index("s") * 2 + lax.axis_index("c")
    base = wid * _B_PER_W
    pltpu.sync_copy(x_hbm.at[wid], idx_v)

    def group(g, carry):
        i0 = g * _NBUF
        descs = []
        for b in range(_NBUF):
            descs.append(
                pltpu.async_copy(
                    table_hbm.at[idx_v.at[i0 + b]], rows_v.at[b], sems.at[b]
                )
            )
        for b in range(_NBUF):
            descs[b].wait()
            pltpu.sync_copy(
                rows_v.at[b],
                out_hbm.at[pl.ds(base + (i0 + b) * _CHUNK, _CHUNK)],
            )
        return carry

    lax.fori_loop(0, _NCHUNK // _NBUF, group, 0)


def kernel(x, table):
    out = _emb_lookup(x.reshape(_NW, _NCHUNK, _CHUNK), table)
    return out.reshape(_BATCH, _SEQ, _D)


# speedup vs baseline: 3.3198x; 1.0038x over previous
import functools

import jax
import jax.numpy as jnp
from jax import lax
from jax.experimental import pallas as pl
from jax.experimental.pallas import tpu as pltpu
from jax.experimental.pallas import tpu_sc as plsc

_BATCH = 4096
_SEQ = 50
_VOCAB = 100000
_D = 128
_B = _BATCH * _SEQ
_NW = 32
_B_PER_W = _B // _NW
_CHUNK = 128
_NCHUNK = _B_PER_W // _CHUNK
_NBUF = 5

_mesh = plsc.VectorSubcoreMesh(core_axis_name="c", subcore_axis_name="s")


@functools.partial(
    pl.kernel,
    mesh=_mesh,
    out_type=jax.ShapeDtypeStruct((_B, _D), jnp.float32),
    scratch_types=[
        pltpu.VMEM((_NCHUNK, _CHUNK), jnp.int32),
        pltpu.VMEM((_NBUF, _CHUNK, _D), jnp.float32),
        pltpu.SemaphoreType.DMA((_NBUF,)),
        pltpu.SemaphoreType.DMA((_NBUF,)),
    ],
)
def _emb_lookup(x_hbm, table_hbm, out_hbm, idx_v, rows_v, gsems, ssems):
    wid = lax.axis_index("s") * 2 + lax.axis_index("c")
    base = wid * _B_PER_W
    pltpu.sync_copy(x_hbm.at[wid], idx_v)

    for b in range(_NBUF):
        pltpu.async_copy(table_hbm.at[idx_v.at[b]], rows_v.at[b], gsems.at[b])

    def group(g, carry):
        i0 = g * _NBUF
        for b in range(_NBUF):
            pltpu.make_async_copy(
                table_hbm.at[pl.ds(0, _CHUNK)], rows_v.at[b], gsems.at[b]
            ).wait()
            pltpu.async_copy(
                rows_v.at[b],
                out_hbm.at[pl.ds(base + (i0 + b) * _CHUNK, _CHUNK)],
                ssems.at[b],
            )
        for b in range(_NBUF):
            pltpu.make_async_copy(
                rows_v.at[b], out_hbm.at[pl.ds(base, _CHUNK)], ssems.at[b]
            ).wait()
            nxt = i0 + _NBUF + b

            @pl.when(nxt < _NCHUNK)
            def _():
                pltpu.async_copy(
                    table_hbm.at[idx_v.at[nxt]], rows_v.at[b], gsems.at[b]
                )

        return carry

    lax.fori_loop(0, _NCHUNK // _NBUF, group, 0)


def kernel(x, table):
    out = _emb_lookup(x.reshape(_NW, _NCHUNK, _CHUNK), table)
    return out.reshape(_BATCH, _SEQ, _D)


# speedup vs baseline: 5.9112x; 1.7806x over previous
import functools

import jax
import jax.numpy as jnp
from jax import lax
from jax.experimental import pallas as pl
from jax.experimental.pallas import tpu as pltpu
from jax.experimental.pallas import tpu_sc as plsc

_BATCH = 4096
_SEQ = 50
_VOCAB = 100000
_D = 128
_NW = 32
_BB_PER_W = _BATCH // _NW
_NBUF = 8

_mesh = plsc.VectorSubcoreMesh(core_axis_name="c", subcore_axis_name="s")


@functools.partial(
    pl.kernel,
    mesh=_mesh,
    out_type=jax.ShapeDtypeStruct((_BATCH, _SEQ, _D), jnp.float32),
    scratch_types=[
        pltpu.VMEM((_BB_PER_W, _SEQ), jnp.int32),
        pltpu.VMEM((_NBUF, _SEQ, _D), jnp.float32),
        pltpu.SemaphoreType.DMA((_NBUF,)),
        pltpu.SemaphoreType.DMA((_NBUF,)),
    ],
)
def _emb_lookup(x_hbm, table_hbm, out_hbm, idx_v, rows_v, gsems, ssems):
    wid = lax.axis_index("s") * 2 + lax.axis_index("c")
    base = wid * _BB_PER_W
    pltpu.sync_copy(x_hbm.at[wid], idx_v)

    for b in range(_NBUF):
        pltpu.async_copy(table_hbm.at[idx_v.at[b]], rows_v.at[b], gsems.at[b])

    def group(g, carry):
        i0 = g * _NBUF
        for b in range(_NBUF):
            pltpu.make_async_copy(
                out_hbm.at[base], rows_v.at[b], gsems.at[b]
            ).wait()
            pltpu.async_copy(
                rows_v.at[b], out_hbm.at[base + i0 + b], ssems.at[b]
            )
        for b in range(_NBUF):
            pltpu.make_async_copy(
                rows_v.at[b], out_hbm.at[base], ssems.at[b]
            ).wait()
            nxt = i0 + _NBUF + b

            @pl.when(nxt < _BB_PER_W)
            def _():
                pltpu.async_copy(
                    table_hbm.at[idx_v.at[nxt]], rows_v.at[b], gsems.at[b]
                )

        return carry

    lax.fori_loop(0, _BB_PER_W // _NBUF, group, 0)


def kernel(x, table):
    return _emb_lookup(x.reshape(_NW, _BB_PER_W, _SEQ), table)


# speedup vs baseline: 5.9314x; 1.0034x over previous
import functools

import jax
import jax.numpy as jnp
from jax import lax
from jax.experimental import pallas as pl
from jax.experimental.pallas import tpu as pltpu
from jax.experimental.pallas import tpu_sc as plsc

_BATCH = 4096
_SEQ = 50
_VOCAB = 100000
_D = 128
_NW = 32
_BB_PER_W = _BATCH // _NW
_NBUF = 8

_mesh = plsc.VectorSubcoreMesh(core_axis_name="c", subcore_axis_name="s")


@functools.partial(
    pl.kernel,
    mesh=_mesh,
    out_type=jax.ShapeDtypeStruct((_BATCH, _SEQ, _D), jnp.float32),
    scratch_types=[
        pltpu.VMEM((_BB_PER_W, _SEQ), jnp.int32),
        pltpu.VMEM((_NBUF, _SEQ, _D), jnp.float32),
        pltpu.SemaphoreType.DMA((_NBUF,)),
        pltpu.SemaphoreType.DMA((_NBUF,)),
    ],
)
def _emb_lookup(x_hbm, table_hbm, out_hbm, idx_v, rows_v, gsems, ssems):
    wid = lax.axis_index("s") * 2 + lax.axis_index("c")
    base = wid * _BB_PER_W
    pltpu.sync_copy(x_hbm.at[pl.ds(base, _BB_PER_W)], idx_v)

    for b in range(_NBUF):
        pltpu.async_copy(table_hbm.at[idx_v.at[b]], rows_v.at[b], gsems.at[b])

    def group(g, carry):
        i0 = g * _NBUF
        for b in range(_NBUF):
            pltpu.make_async_copy(
                out_hbm.at[base], rows_v.at[b], gsems.at[b]
            ).wait()
            pltpu.async_copy(
                rows_v.at[b], out_hbm.at[base + i0 + b], ssems.at[b]
            )
        for b in range(_NBUF):
            pltpu.make_async_copy(
                rows_v.at[b], out_hbm.at[base], ssems.at[b]
            ).wait()
            nxt = i0 + _NBUF + b

            @pl.when(nxt < _BB_PER_W)
            def _():
                pltpu.async_copy(
                    table_hbm.at[idx_v.at[nxt]], rows_v.at[b], gsems.at[b]
                )

        return carry

    lax.fori_loop(0, _BB_PER_W // _NBUF, group, 0)


def kernel(x, table):
    return _emb_lookup(x, table)


# speedup vs baseline: 10.4334x; 1.7590x over previous
import functools

import jax
import jax.numpy as jnp
from jax import lax
from jax.experimental import pallas as pl
from jax.experimental.pallas import tpu as pltpu
from jax.experimental.pallas import tpu_sc as plsc

_BATCH = 4096
_SEQ = 50
_VOCAB = 100000
_D = 128
_NW = 32
_BB_PER_W = _BATCH // _NW
_NBUF = 5

_mesh = plsc.VectorSubcoreMesh(core_axis_name="c", subcore_axis_name="s")


@functools.partial(
    pl.kernel,
    mesh=_mesh,
    out_type=jax.ShapeDtypeStruct((_SEQ, _BATCH, _D), jnp.float32),
    scratch_types=[
        pltpu.VMEM((_SEQ, _BB_PER_W), jnp.int32),
        pltpu.VMEM((_NBUF, _BB_PER_W, _D), jnp.float32),
        pltpu.SemaphoreType.DMA((_NBUF,)),
        pltpu.SemaphoreType.DMA((_NBUF,)),
    ],
)
def _emb_lookup(xt_hbm, table_hbm, out_hbm, idxt_v, rows_v, gsems, ssems):
    wid = lax.axis_index("s") * 2 + lax.axis_index("c")
    b0 = wid * _BB_PER_W
    pltpu.sync_copy(xt_hbm.at[:, pl.ds(b0, _BB_PER_W)], idxt_v)

    for b in range(_NBUF):
        pltpu.async_copy(table_hbm.at[idxt_v.at[b]], rows_v.at[b], gsems.at[b])

    def group(g, carry):
        i0 = g * _NBUF
        for b in range(_NBUF):
            pltpu.make_async_copy(
                out_hbm.at[0, pl.ds(0, _BB_PER_W)], rows_v.at[b], gsems.at[b]
            ).wait()
            pltpu.async_copy(
                rows_v.at[b],
                out_hbm.at[i0 + b, pl.ds(b0, _BB_PER_W)],
                ssems.at[b],
            )
        for b in range(_NBUF):
            pltpu.make_async_copy(
                rows_v.at[b], out_hbm.at[0, pl.ds(0, _BB_PER_W)], ssems.at[b]
            ).wait()
            nxt = i0 + _NBUF + b

            @pl.when(nxt < _SEQ)
            def _():
                pltpu.async_copy(
                    table_hbm.at[idxt_v.at[nxt]], rows_v.at[b], gsems.at[b]
                )

        return carry

    lax.fori_loop(0, _SEQ // _NBUF, group, 0)


def kernel(x, table):
    out = _emb_lookup(x.T, table)
    return jnp.transpose(out, (1, 0, 2))


# speedup vs baseline: 10.6935x; 1.0249x over previous
import functools

import jax
import jax.numpy as jnp
from jax import lax
from jax.experimental import pallas as pl
from jax.experimental.pallas import tpu as pltpu
from jax.experimental.pallas import tpu_sc as plsc

_BATCH = 4096
_SEQ = 50
_VOCAB = 100000
_D = 128
_NW = 32
_BB_PER_W = _BATCH // _NW
_NCH = 2
_CH = _BB_PER_W // _NCH
_NBUF = 10

_mesh = plsc.VectorSubcoreMesh(core_axis_name="c", subcore_axis_name="s")


@functools.partial(
    pl.kernel,
    mesh=_mesh,
    out_type=jax.ShapeDtypeStruct((_SEQ, _BATCH, _D), jnp.float32),
    scratch_types=[
        pltpu.VMEM((_SEQ, _BB_PER_W), jnp.int32),
        pltpu.VMEM((_NBUF, _CH, _D), jnp.float32),
        pltpu.SemaphoreType.DMA((_NBUF,)),
        pltpu.SemaphoreType.DMA((_NBUF,)),
    ],
)
def _emb_lookup(xt_hbm, table_hbm, out_hbm, idxt_v, rows_v, gsems, ssems):
    wid = lax.axis_index("s") * 2 + lax.axis_index("c")
    b0 = wid * _BB_PER_W
    pltpu.sync_copy(xt_hbm.at[:, pl.ds(b0, _BB_PER_W)], idxt_v)

    def issue_gather(c, b):
        s, half = c // _NCH, c % _NCH
        pltpu.async_copy(
            table_hbm.at[idxt_v.at[s, pl.ds(half * _CH, _CH)]],
            rows_v.at[b],
            gsems.at[b],
        )

    for b in range(_NBUF):
        issue_gather(b, b)

    def group(g, carry):
        i0 = g * _NBUF
        for b in range(_NBUF):
            c = i0 + b
            s, half = c // _NCH, c % _NCH
            pltpu.make_async_copy(
                out_hbm.at[0, pl.ds(0, _CH)], rows_v.at[b], gsems.at[b]
            ).wait()
            pltpu.async_copy(
                rows_v.at[b],
                out_hbm.at[s, pl.ds(b0 + half * _CH, _CH)],
                ssems.at[b],
            )
        for b in range(_NBUF):
            pltpu.make_async_copy(
                rows_v.at[b], out_hbm.at[0, pl.ds(0, _CH)], ssems.at[b]
            ).wait()
            nxt = i0 + _NBUF + b

            @pl.when(nxt < _SEQ * _NCH)
            def _():
                issue_gather(nxt, b)

        return carry

    lax.fori_loop(0, _SEQ * _NCH // _NBUF, group, 0)


def kernel(x, table):
    out = _emb_lookup(x.T, table)
    return jnp.transpose(out, (1, 0, 2))
